# Initial kernel scaffold; baseline (speedup 1.0000x reference)
import jax
import jax.numpy as jnp
from jax.experimental import pallas as pl


def kernel(x, table, W1, b1, W2, b2, W3, b3):
    raise NotImplementedError("write your pallas kernel here")


# speedup vs baseline: 9.6934x; 9.6934x over previous
import functools

import jax
import jax.numpy as jnp
from jax import lax
from jax.experimental import pallas as pl
from jax.experimental.pallas import tpu as pltpu
from jax.experimental.pallas import tpu_sc as plsc

B = 16384
BAG = 100
E = 64
H1, H2 = 512, 256

NC, NS, L = 2, 16, 16
NW = NC * NS
BPW = B // NW
CH = 8
NR = BPW // CH
NCOL = E // L


def _pool_sc(x, table):
    mesh = plsc.VectorSubcoreMesh(
        core_axis_name="c", subcore_axis_name="s",
        num_cores=NC, num_subcores=NS)

    @functools.partial(
        pl.kernel,
        out_type=jax.ShapeDtypeStruct((B, E), jnp.float32),
        mesh=mesh,
        compiler_params=pltpu.CompilerParams(use_tc_tiling_on_sc=False),
        scratch_types=[
            pltpu.VMEM((CH, BAG), jnp.int32),
            pltpu.VMEM((CH * BAG, E), jnp.float32),
            pltpu.VMEM((CH, E), jnp.float32),
            pltpu.SemaphoreType.DMA,
        ],
    )
    def pool(x_hbm, table_hbm, out_hbm, idx_v, rows_v, obuf_v, gsem):
        wid = lax.axis_index("s") * NC + lax.axis_index("c")
        base = wid * BPW

        def round_body(r, carry):
            bag0 = base + r * CH
            pltpu.sync_copy(x_hbm.at[pl.ds(bag0, CH)], idx_v)
            for j in range(CH):
                pltpu.async_copy(
                    table_hbm.at[idx_v.at[j]],
                    rows_v.at[pl.ds(j * BAG, BAG)], gsem)
            pltpu.make_async_copy(
                table_hbm.at[pl.ds(0, CH * BAG)], rows_v, gsem).wait()
            for j in range(CH):
                def acc_body(rr, accs, j=j):
                    row = j * BAG + rr
                    return tuple(
                        accs[c] + rows_v[row, pl.ds(c * L, L)]
                        for c in range(NCOL))
                accs = lax.fori_loop(
                    0, BAG, acc_body,
                    tuple(jnp.zeros((L,), jnp.float32) for _ in range(NCOL)))
                for c in range(NCOL):
                    obuf_v[j, pl.ds(c * L, L)] = accs[c]
            pltpu.sync_copy(obuf_v, out_hbm.at[pl.ds(bag0, CH)])
            return carry

        lax.fori_loop(0, NR, round_body, 0)

    return pool(x, table)


BM = 1024


def _mlp_body(x_ref, w1_ref, b1_ref, w2_ref, b2_ref, w3t_ref, b3_ref, o_ref):
    xr = jnp.maximum(x_ref[...], 0.0) * (1.0 / BAG)
    h1 = jnp.dot(xr, w1_ref[...], preferred_element_type=jnp.float32)
    h1 = jnp.maximum(h1 + b1_ref[...], 0.0)
    h2 = jnp.dot(h1, w2_ref[...], preferred_element_type=jnp.float32)
    h2 = jnp.maximum(h2 + b2_ref[...], 0.0)
    o_ref[...] = jnp.sum(h2 * w3t_ref[...], axis=1) + b3_ref[0]


def _mlp_tc(pooled, W1, b1, W2, b2, W3, b3):
    grid = (B // BM,)
    return pl.pallas_call(
        _mlp_body,
        grid=grid,
        in_specs=[
            pl.BlockSpec((BM, E), lambda i: (i, 0)),
            pl.BlockSpec((E, H1), lambda i: (0, 0)),
            pl.BlockSpec((1, H1), lambda i: (0, 0)),
            pl.BlockSpec((H1, H2), lambda i: (0, 0)),
            pl.BlockSpec((1, H2), lambda i: (0, 0)),
            pl.BlockSpec((1, H2), lambda i: (0, 0)),
            pl.BlockSpec(memory_space=pltpu.SMEM),
        ],
        out_specs=pl.BlockSpec((BM,), lambda i: (i,)),
        out_shape=jax.ShapeDtypeStruct((B,), jnp.float32),
    )(pooled, W1, b1.reshape(1, H1), W2, b2.reshape(1, H2),
      W3.reshape(1, H2), b3)


def kernel(x, table, W1, b1, W2, b2, W3, b3):
    pooled = _pool_sc(x, table)
    return _mlp_tc(pooled, W1, b1, W2, b2, W3, b3)
